# baseline (speedup 1.0000x reference)
import functools

import jax
import jax.numpy as jnp
from jax import lax
from jax.experimental import pallas as pl
from jax.experimental.pallas import tpu as pltpu
from jax.experimental.pallas import tpu_sc as plsc

_B = 4
_S = 2048
_E = 128
_EPS = 1e-12
_L = 16
_NE = _E // _L

_NC = 2
_NS = 16
_NW = _NC * _NS
_NTOK = _B * _S
_TPW = _NTOK // _NW
_ICH = 128
_NCH = _TPW // _ICH

_HT = _ICH // 2
_HG = _HT // _L

_INV_E = 1.0 / _E
_MAGIC = 0x5F3759DF


def _tree_sum(vs):
    while len(vs) > 1:
        vs = [a + b for a, b in zip(vs[::2], vs[1::2])]
    return vs[0]


def _fused_sc(ids, tt, wemb, pemb, temb, gamma, beta):
    mesh = plsc.VectorSubcoreMesh(core_axis_name="c", subcore_axis_name="s")

    @functools.partial(
        pl.kernel,
        out_type=jax.ShapeDtypeStruct((_B, _S, _E), jnp.float32),
        mesh=mesh,
        compiler_params=pltpu.CompilerParams(needs_layout_passes=False),
        scratch_types=[
            pltpu.VMEM((_NCH, _ICH), jnp.int32),
            pltpu.VMEM((_NCH, _ICH), jnp.int32),
            pltpu.VMEM((_TPW, _E), jnp.float32),
            pltpu.VMEM((_ICH, _E), jnp.float32),
            pltpu.VMEM((2, _E), jnp.float32),
            pltpu.VMEM((2, _E), jnp.float32),
            pltpu.SemaphoreType.DMA,
            pltpu.SemaphoreType.DMA,
            pltpu.SemaphoreType.DMA,
            pltpu.SemaphoreType.DMA,
        ],
    )
    def k(ids_hbm, tt_hbm, wemb_hbm, pemb_hbm, temb_hbm, g_hbm, b_hbm,
          out_hbm, idx_v, tt_v, rows_v, pos_v, te_v, gb_v, sem_a, sem_g0,
          sem_g1, sem_o):
        col = lax.axis_index("s") * _ICH
        b0 = lax.axis_index("c") * _NCH

        aux = [
            pltpu.async_copy(pemb_hbm.at[pl.ds(col, _ICH)], pos_v, sem_a),
            pltpu.async_copy(temb_hbm, te_v, sem_a),
            pltpu.async_copy(g_hbm, gb_v.at[0], sem_a),
            pltpu.async_copy(b_hbm, gb_v.at[1], sem_a),
        ]
        for j in range(_NCH):
            aux.append(pltpu.async_copy(
                tt_hbm.at[b0 + j, pl.ds(col, _ICH)], tt_v.at[j], sem_a))
            pltpu.sync_copy(ids_hbm.at[b0 + j, pl.ds(col, _ICH)],
                            idx_v.at[j])
        gsems = [sem_g0, sem_g1]
        gathers = [
            pltpu.async_copy(wemb_hbm.at[idx_v.at[j]],
                             rows_v.at[pl.ds(j * _ICH, _ICH)], gsems[j])
            for j in range(_NCH)
        ]
        for cp in aux:
            cp.wait()

        def compute_half(j, h):
            lo = j * _ICH

            @plsc.parallel_loop(h * _HG, (h + 1) * _HG, unroll=1)
            def _(g):
                tlf = tt_v[j, pl.ds(g * _L, _L)].astype(jnp.float32)
                r0 = [te_v[0, pl.ds(e * _L, _L)] for e in range(_NE)]
                dd = [te_v[1, pl.ds(e * _L, _L)] - r0[e] for e in range(_NE)]
                gg = [gb_v[0, pl.ds(e * _L, _L)] for e in range(_NE)]
                bb = [gb_v[1, pl.ds(e * _L, _L)] for e in range(_NE)]
                for jj in range(_L):
                    p_i = g * _L + jj
                    i = lo + p_i
                    tf = jnp.full((_L,), tlf[jj])
                    s = []
                    for e in range(_NE):
                        w = rows_v[i, pl.ds(e * _L, _L)]
                        p = pos_v[p_i, pl.ds(e * _L, _L)]
                        s.append(w + p + r0[e] + tf * dd[e])
                    tot = _tree_sum(s)
                    sq = _tree_sum([x * x for x in s])
                    mean = jnp.sum(tot) * _INV_E
                    var = jnp.sum(sq) * _INV_E - mean * mean + _EPS
                    vv = jnp.full((_L,), var)
                    iv = plsc.bitcast(vv, jnp.int32)
                    y = plsc.bitcast(jnp.int32(_MAGIC) - (iv >> 1),
                                     jnp.float32)
                    h = 0.5 * vv
                    for _ in range(3):
                        y = y * (1.5 - h * y * y)
                    mv = jnp.full((_L,), mean)
                    for e in range(_NE):
                        rows_v[i, pl.ds(e * _L, _L)] = (
                            (s[e] - mv) * y * gg[e] + bb[e])

        outs = []
        for j in range(_NCH):
            gathers[j].wait()
            for h in range(2):
                compute_half(j, h)
                outs.append(pltpu.async_copy(
                    rows_v.at[pl.ds(j * _ICH + h * _HT, _HT)],
                    out_hbm.at[b0 + j, pl.ds(col + h * _HT, _HT)], sem_o))
        for cp in outs:
            cp.wait()

    return k(ids, tt, wemb, pemb, temb, gamma, beta)


def kernel(input_ids, token_type_ids, word_embeddings, position_embeddings,
           token_type_embeddings, gamma, beta):
    return _fused_sc(input_ids.astype(jnp.int32),
                     token_type_ids.astype(jnp.int32),
                     word_embeddings, position_embeddings,
                     token_type_embeddings, gamma, beta)


# speedup vs baseline: 1.0875x; 1.0875x over previous
import functools

import jax
import jax.numpy as jnp
from jax import lax
from jax.experimental import pallas as pl
from jax.experimental.pallas import tpu as pltpu
from jax.experimental.pallas import tpu_sc as plsc

_B = 4
_S = 2048
_E = 128
_EPS = 1e-12
_L = 16
_NE = _E // _L

_NC = 2
_NS = 16
_NW = _NC * _NS
_NTOK = _B * _S
_TPW = _NTOK // _NW
_ICH = 128
_NCH = _TPW // _ICH

_INV_E = 1.0 / _E
_MAGIC = 0x5F3759DF


def _tree_sum(vs):
    while len(vs) > 1:
        vs = [a + b for a, b in zip(vs[::2], vs[1::2])]
    return vs[0]


def _fused_sc(ids, tt, wemb, pemb, temb, gamma, beta):
    mesh = plsc.VectorSubcoreMesh(core_axis_name="c", subcore_axis_name="s")

    @functools.partial(
        pl.kernel,
        out_type=jax.ShapeDtypeStruct((_B, _S, _E), jnp.float32),
        mesh=mesh,
        compiler_params=pltpu.CompilerParams(needs_layout_passes=False),
        scratch_types=[
            pltpu.VMEM((_NCH, _ICH), jnp.int32),
            pltpu.VMEM((_NCH, _ICH), jnp.int32),
            pltpu.VMEM((_TPW, _E), jnp.float32),
            pltpu.VMEM((_ICH, _E), jnp.float32),
            pltpu.VMEM((2, _E), jnp.float32),
            pltpu.VMEM((2, _E), jnp.float32),
            pltpu.SemaphoreType.DMA,
            pltpu.SemaphoreType.DMA,
            pltpu.SemaphoreType.DMA,
            pltpu.SemaphoreType.DMA,
        ],
    )
    def k(ids_hbm, tt_hbm, wemb_hbm, pemb_hbm, temb_hbm, g_hbm, b_hbm,
          out_hbm, idx_v, tt_v, rows_v, pos_v, te_v, gb_v, sem_a, sem_g0,
          sem_g1, sem_o):
        col = lax.axis_index("s") * _ICH
        b0 = lax.axis_index("c") * _NCH

        for j in range(_NCH):
            pltpu.sync_copy(ids_hbm.at[b0 + j, pl.ds(col, _ICH)],
                            idx_v.at[j])
        gsems = [sem_g0, sem_g1]
        gathers = [
            pltpu.async_copy(wemb_hbm.at[idx_v.at[j]],
                             rows_v.at[pl.ds(j * _ICH, _ICH)], gsems[j])
            for j in range(_NCH)
        ]
        aux = [
            pltpu.async_copy(pemb_hbm.at[pl.ds(col, _ICH)], pos_v, sem_a),
            pltpu.async_copy(temb_hbm, te_v, sem_a),
            pltpu.async_copy(g_hbm, gb_v.at[0], sem_a),
            pltpu.async_copy(b_hbm, gb_v.at[1], sem_a),
        ]
        for j in range(_NCH):
            aux.append(pltpu.async_copy(
                tt_hbm.at[b0 + j, pl.ds(col, _ICH)], tt_v.at[j], sem_a))
        for cp in aux:
            cp.wait()

        def compute_chunk(j):
            lo = j * _ICH

            @plsc.parallel_loop(0, _ICH // _L, unroll=1)
            def _(g):
                tlf = tt_v[j, pl.ds(g * _L, _L)].astype(jnp.float32)
                r0 = [te_v[0, pl.ds(e * _L, _L)] for e in range(_NE)]
                dd = [te_v[1, pl.ds(e * _L, _L)] - r0[e] for e in range(_NE)]
                gg = [gb_v[0, pl.ds(e * _L, _L)] for e in range(_NE)]
                bb = [gb_v[1, pl.ds(e * _L, _L)] for e in range(_NE)]
                for jj in range(_L):
                    p_i = g * _L + jj
                    i = lo + p_i
                    tf = jnp.full((_L,), tlf[jj])
                    s = []
                    for e in range(_NE):
                        w = rows_v[i, pl.ds(e * _L, _L)]
                        p = pos_v[p_i, pl.ds(e * _L, _L)]
                        s.append(w + p + r0[e] + tf * dd[e])
                    tot = _tree_sum(s)
                    sq = _tree_sum([x * x for x in s])
                    mean = jnp.sum(tot) * _INV_E
                    var = jnp.sum(sq) * _INV_E - mean * mean + _EPS
                    vv = jnp.full((_L,), var)
                    iv = plsc.bitcast(vv, jnp.int32)
                    y = plsc.bitcast(jnp.int32(_MAGIC) - (iv >> 1),
                                     jnp.float32)
                    h = 0.5 * vv
                    for _ in range(3):
                        y = y * (1.5 - h * y * y)
                    mv = jnp.full((_L,), mean)
                    for e in range(_NE):
                        rows_v[i, pl.ds(e * _L, _L)] = (
                            (s[e] - mv) * y * gg[e] + bb[e])

        gathers[0].wait()
        compute_chunk(0)
        out0 = pltpu.async_copy(
            rows_v.at[pl.ds(0, _ICH)],
            out_hbm.at[b0, pl.ds(col, _ICH)], sem_o)
        gathers[1].wait()
        compute_chunk(1)
        out1 = pltpu.async_copy(
            rows_v.at[pl.ds(_ICH, _ICH)],
            out_hbm.at[b0 + 1, pl.ds(col, _ICH)], sem_o)
        out0.wait()
        out1.wait()

    return k(ids, tt, wemb, pemb, temb, gamma, beta)


def kernel(input_ids, token_type_ids, word_embeddings, position_embeddings,
           token_type_embeddings, gamma, beta):
    return _fused_sc(input_ids.astype(jnp.int32),
                     token_type_ids.astype(jnp.int32),
                     word_embeddings, position_embeddings,
                     token_type_embeddings, gamma, beta)


# speedup vs baseline: 1.1266x; 1.0360x over previous
import functools

import jax
import jax.numpy as jnp
from jax import lax
from jax.experimental import pallas as pl
from jax.experimental.pallas import tpu as pltpu
from jax.experimental.pallas import tpu_sc as plsc

_B = 4
_S = 2048
_E = 128
_EPS = 1e-12
_L = 16
_NE = _E // _L

_NC = 2
_NS = 16
_NW = _NC * _NS
_NTOK = _B * _S
_TPW = _NTOK // _NW
_ICH = 128
_NCH = _TPW // _ICH

_INV_E = 1.0 / _E
_MAGIC = 0x5F3759DF


def _tree_sum(vs):
    while len(vs) > 1:
        vs = [a + b for a, b in zip(vs[::2], vs[1::2])]
    return vs[0]


def _fused_sc(ids, tt, wemb, pemb, temb, gamma, beta):
    mesh = plsc.VectorSubcoreMesh(core_axis_name="c", subcore_axis_name="s")

    @functools.partial(
        pl.kernel,
        out_type=jax.ShapeDtypeStruct((_B, _S, _E), jnp.float32),
        mesh=mesh,
        compiler_params=pltpu.CompilerParams(needs_layout_passes=False),
        scratch_types=[
            pltpu.VMEM((_NCH, _ICH), jnp.int32),
            pltpu.VMEM((_NCH, _ICH), jnp.int32),
            pltpu.VMEM((_TPW, _E), jnp.float32),
            pltpu.VMEM((_ICH, _E), jnp.float32),
            pltpu.VMEM((2, _E), jnp.float32),
            pltpu.VMEM((2, _E), jnp.float32),
            pltpu.SemaphoreType.DMA,
            pltpu.SemaphoreType.DMA,
            pltpu.SemaphoreType.DMA,
            pltpu.SemaphoreType.DMA,
        ],
    )
    def k(ids_hbm, tt_hbm, wemb_hbm, pemb_hbm, temb_hbm, g_hbm, b_hbm,
          out_hbm, idx_v, tt_v, rows_v, pos_v, te_v, gb_v, sem_a, sem_g0,
          sem_g1, sem_o):
        col = lax.axis_index("s") * _ICH
        b0 = lax.axis_index("c") * _NCH

        aux = [
            pltpu.async_copy(pemb_hbm.at[pl.ds(col, _ICH)], pos_v, sem_a),
            pltpu.async_copy(temb_hbm, te_v, sem_a),
            pltpu.async_copy(g_hbm, gb_v.at[0], sem_a),
            pltpu.async_copy(b_hbm, gb_v.at[1], sem_a),
        ]
        gsems = [sem_g0, sem_g1]
        gathers = []
        for j in range(_NCH):
            aux.append(pltpu.async_copy(
                tt_hbm.at[b0 + j, pl.ds(col, _ICH)], tt_v.at[j], sem_a))
            pltpu.sync_copy(ids_hbm.at[b0 + j, pl.ds(col, _ICH)],
                            idx_v.at[j])
            gathers.append(pltpu.async_copy(
                wemb_hbm.at[idx_v.at[j]],
                rows_v.at[pl.ds(j * _ICH, _ICH)], gsems[j]))
        for cp in aux:
            cp.wait()

        def compute_chunk(j):
            lo = j * _ICH

            @plsc.parallel_loop(0, _ICH // _L, unroll=1)
            def _(g):
                tlf = tt_v[j, pl.ds(g * _L, _L)].astype(jnp.float32)
                r0 = [te_v[0, pl.ds(e * _L, _L)] for e in range(_NE)]
                dd = [te_v[1, pl.ds(e * _L, _L)] - r0[e] for e in range(_NE)]
                gg = [gb_v[0, pl.ds(e * _L, _L)] for e in range(_NE)]
                bb = [gb_v[1, pl.ds(e * _L, _L)] for e in range(_NE)]
                for jj in range(_L):
                    p_i = g * _L + jj
                    i = lo + p_i
                    tf = jnp.full((_L,), tlf[jj])
                    s = []
                    for e in range(_NE):
                        w = rows_v[i, pl.ds(e * _L, _L)]
                        p = pos_v[p_i, pl.ds(e * _L, _L)]
                        s.append(w + p + r0[e] + tf * dd[e])
                    tot = _tree_sum(s)
                    sq = _tree_sum([x * x for x in s])
                    mean = jnp.sum(tot) * _INV_E
                    var = jnp.sum(sq) * _INV_E - mean * mean + _EPS
                    vv = jnp.full((_L,), var)
                    iv = plsc.bitcast(vv, jnp.int32)
                    y = plsc.bitcast(jnp.int32(_MAGIC) - (iv >> 1),
                                     jnp.float32)
                    h = 0.5 * vv
                    for _ in range(3):
                        y = y * (1.5 - h * y * y)
                    mv = jnp.full((_L,), mean)
                    for e in range(_NE):
                        rows_v[i, pl.ds(e * _L, _L)] = (
                            (s[e] - mv) * y * gg[e] + bb[e])

        gathers[0].wait()
        compute_chunk(0)
        out0 = pltpu.async_copy(
            rows_v.at[pl.ds(0, _ICH)],
            out_hbm.at[b0, pl.ds(col, _ICH)], sem_o)
        gathers[1].wait()
        compute_chunk(1)
        out1 = pltpu.async_copy(
            rows_v.at[pl.ds(_ICH, _ICH)],
            out_hbm.at[b0 + 1, pl.ds(col, _ICH)], sem_o)
        out0.wait()
        out1.wait()

    return k(ids, tt, wemb, pemb, temb, gamma, beta)


def kernel(input_ids, token_type_ids, word_embeddings, position_embeddings,
           token_type_embeddings, gamma, beta):
    return _fused_sc(input_ids.astype(jnp.int32),
                     token_type_ids.astype(jnp.int32),
                     word_embeddings, position_embeddings,
                     token_type_embeddings, gamma, beta)
